# baseline (speedup 1.0000x reference)
import functools

import jax
import jax.numpy as jnp
from jax import lax
from jax.experimental import pallas as pl
from jax.experimental.pallas import tpu as pltpu
from jax.experimental.pallas import tpu_sc as plsc

N = 10000
E = 320000
H = 128

NC = 2
NT = 16
K = 80
NB = 256
EP = NT * NB * K
NQ = 8
NBC = NB // NQ
NP = 10240
RPT = NP // NT

ROW_BLK = 2000


def _pre_body(h_ref, c_ref, wt_ref, b_ref, p_ref):
    g = jax.nn.sigmoid(
        jnp.dot(h_ref[...], wt_ref[...], preferred_element_type=jnp.float32)
        + b_ref[...])
    p_ref[...] = g * c_ref[...]


_pre = pl.pallas_call(
    _pre_body,
    grid=(N // ROW_BLK,),
    in_specs=[
        pl.BlockSpec((ROW_BLK, H), lambda i: (i, 0)),
        pl.BlockSpec((ROW_BLK, H), lambda i: (i, 0)),
        pl.BlockSpec((H, H), lambda i: (0, 0)),
        pl.BlockSpec((1, H), lambda i: (0, 0)),
    ],
    out_specs=pl.BlockSpec((ROW_BLK, H), lambda i: (i, 0)),
    out_shape=jax.ShapeDtypeStruct((N, H), jnp.float32),
)


def _post_body(agg_ref, wt_ref, b_ref, h_ref, c_ref):
    ht = agg_ref[0]
    ca = agg_ref[1]
    iou = (jnp.dot(ht, wt_ref[...], preferred_element_type=jnp.float32)
           + b_ref[...])
    i = jax.nn.sigmoid(iou[:, :H])
    o = jax.nn.sigmoid(iou[:, H:2 * H])
    u = jnp.tanh(iou[:, 2 * H:])
    c_new = i * u + ca
    h_ref[...] = o * jnp.tanh(c_new)
    c_ref[...] = c_new


_post = pl.pallas_call(
    _post_body,
    grid=(N // ROW_BLK,),
    in_specs=[
        pl.BlockSpec((NC, ROW_BLK, H), lambda i: (0, i, 0)),
        pl.BlockSpec((H, 3 * H), lambda i: (0, 0)),
        pl.BlockSpec((1, 3 * H), lambda i: (0, 0)),
    ],
    out_specs=[
        pl.BlockSpec((ROW_BLK, H), lambda i: (i, 0)),
        pl.BlockSpec((ROW_BLK, H), lambda i: (i, 0)),
    ],
    out_shape=[
        jax.ShapeDtypeStruct((N, H), jnp.float32),
        jax.ShapeDtypeStruct((N, H), jnp.float32),
    ],
)


def _edge_body(tabh, tabp, src3, dst3, out, src_v, dst_v, rows_v, acc, sem):
    c = lax.axis_index("c")
    s = lax.axis_index("s")

    zero16 = jnp.zeros((16,), jnp.float32)

    def _zrow(i, carry):
        for j in range(H // 16):
            rows_v[i, 16 * j:16 * (j + 1)] = zero16
        return carry

    lax.fori_loop(0, K, _zrow, 0)
    base = s * RPT
    for t in range(RPT // K):
        pltpu.sync_copy(rows_v, acc.at[pl.ds(base + K * t, K)])
    plsc.subcore_barrier()

    def _loop(tab):
        def _chunk(q, carry):
            pltpu.sync_copy(src3.at[s, q], src_v)
            pltpu.sync_copy(dst3.at[s, q], dst_v)

            def _blk(j, carry2):
                pltpu.async_copy(tab.at[src_v.at[j]], rows_v, sem).wait()
                pltpu.sync_copy(rows_v, acc.at[dst_v.at[j]], add=True)
                return carry2

            lax.fori_loop(0, NBC, _blk, 0)
            return carry

        lax.fori_loop(0, NQ, _chunk, 0)

    @pl.when(c == 0)
    def _():
        _loop(tabh)

    @pl.when(c == 1)
    def _():
        _loop(tabp)

    plsc.subcore_barrier()

    for t in range(RPT // K):
        pltpu.sync_copy(acc.at[pl.ds(base + K * t, K)], rows_v)
        pltpu.sync_copy(rows_v, out.at[c, pl.ds(base + K * t, K)])


@functools.lru_cache(maxsize=1)
def _edge_kernel():
    return pl.kernel(
        _edge_body,
        out_type=pltpu.HBM((NC, NP, H), jnp.float32),
        mesh=plsc.VectorSubcoreMesh(core_axis_name="c", subcore_axis_name="s"),
        scratch_types=[
            pltpu.VMEM((NBC, K), jnp.int32),
            pltpu.VMEM((NBC, K), jnp.int32),
            pltpu.VMEM((K, H), jnp.float32),
            pltpu.VMEM_SHARED((NP, H), jnp.float32),
            pltpu.SemaphoreType.DMA,
        ],
    )


@jax.jit
def kernel(h, c, edge_index, U_iou_w, U_f_w, U_f_b, b_iou):
    src = edge_index[0].astype(jnp.int32)
    dst = edge_index[1].astype(jnp.int32)

    p = _pre(h, c, U_f_w.T, U_f_b.reshape(1, H))

    pad = EP - E
    src3 = jnp.concatenate([src, jnp.zeros((pad,), jnp.int32)])
    dst3 = jnp.concatenate([dst, jnp.full((pad,), N, jnp.int32)])
    src3 = src3.reshape(NT, NQ, NBC, K)
    dst3 = dst3.reshape(NT, NQ, NBC, K)

    agg = _edge_kernel()(h, p, src3, dst3)
    h_new, c_new = _post(agg, U_iou_w.T, b_iou)
    return h_new, c_new


# speedup vs baseline: 1.0140x; 1.0140x over previous
import functools

import jax
import jax.numpy as jnp
from jax import lax
from jax.experimental import pallas as pl
from jax.experimental.pallas import tpu as pltpu
from jax.experimental.pallas import tpu_sc as plsc

N = 10000
E = 320000
H = 128

NC = 2
NT = 16
K = 80
NB = 256
EP = NT * NB * K
NQ = 8
NBC = NB // NQ
NP = 10240
RPT = NP // NT

ROW_BLK = 2000


def _pre_body(h_ref, c_ref, wt_ref, b_ref, p_ref):
    g = jax.nn.sigmoid(
        jnp.dot(h_ref[...], wt_ref[...], preferred_element_type=jnp.float32)
        + b_ref[...])
    p_ref[...] = g * c_ref[...]


_pre = pl.pallas_call(
    _pre_body,
    grid=(N // ROW_BLK,),
    in_specs=[
        pl.BlockSpec((ROW_BLK, H), lambda i: (i, 0)),
        pl.BlockSpec((ROW_BLK, H), lambda i: (i, 0)),
        pl.BlockSpec((H, H), lambda i: (0, 0)),
        pl.BlockSpec((1, H), lambda i: (0, 0)),
    ],
    out_specs=pl.BlockSpec((ROW_BLK, H), lambda i: (i, 0)),
    out_shape=jax.ShapeDtypeStruct((N, H), jnp.float32),
)


def _post_body(agg_ref, wt_ref, b_ref, h_ref, c_ref):
    ht = agg_ref[0]
    ca = agg_ref[1]
    iou = (jnp.dot(ht, wt_ref[...], preferred_element_type=jnp.float32)
           + b_ref[...])
    i = jax.nn.sigmoid(iou[:, :H])
    o = jax.nn.sigmoid(iou[:, H:2 * H])
    u = jnp.tanh(iou[:, 2 * H:])
    c_new = i * u + ca
    h_ref[...] = o * jnp.tanh(c_new)
    c_ref[...] = c_new


_post = pl.pallas_call(
    _post_body,
    grid=(N // ROW_BLK,),
    in_specs=[
        pl.BlockSpec((NC, ROW_BLK, H), lambda i: (0, i, 0)),
        pl.BlockSpec((H, 3 * H), lambda i: (0, 0)),
        pl.BlockSpec((1, 3 * H), lambda i: (0, 0)),
    ],
    out_specs=[
        pl.BlockSpec((ROW_BLK, H), lambda i: (i, 0)),
        pl.BlockSpec((ROW_BLK, H), lambda i: (i, 0)),
    ],
    out_shape=[
        jax.ShapeDtypeStruct((N, H), jnp.float32),
        jax.ShapeDtypeStruct((N, H), jnp.float32),
    ],
)


def _edge_body(tabh, tabp, src3, dst3, out, src_v, dst_v, rows_v, acc, sem):
    c = lax.axis_index("c")
    s = lax.axis_index("s")

    zero16 = jnp.zeros((16,), jnp.float32)

    def _zrow(i, carry):
        for j in range(H // 16):
            rows_v[i, 16 * j:16 * (j + 1)] = zero16
        return carry

    lax.fori_loop(0, K, _zrow, 0)
    base = s * RPT
    for t in range(RPT // K):
        pltpu.sync_copy(rows_v, acc.at[pl.ds(base + K * t, K)])
    plsc.subcore_barrier()

    def _loop(tab):
        def _chunk(q, carry):
            pltpu.sync_copy(src3.at[s, q], src_v)
            pltpu.sync_copy(dst3.at[s, q], dst_v)

            def _blk(j, carry2):
                pltpu.async_copy(tab.at[src_v.at[j]], rows_v, sem).wait()
                pltpu.sync_copy(rows_v, acc.at[dst_v.at[j]], add=True)
                return carry2

            lax.fori_loop(0, NBC, _blk, 0)
            return carry

        lax.fori_loop(0, NQ, _chunk, 0)

    @pl.when(c == 0)
    def _():
        _loop(tabh)

    @pl.when(c == 1)
    def _():
        _loop(tabp)

    plsc.subcore_barrier()

    for t in range(RPT // K):
        pltpu.sync_copy(acc.at[pl.ds(base + K * t, K)], rows_v)
        pltpu.sync_copy(rows_v, out.at[c, pl.ds(base + K * t, K)])


@functools.lru_cache(maxsize=1)
def _edge_kernel():
    return pl.kernel(
        _edge_body,
        out_type=pltpu.HBM((NC, NP, H), jnp.float32),
        mesh=plsc.VectorSubcoreMesh(core_axis_name="c", subcore_axis_name="s"),
        scratch_types=[
            pltpu.VMEM((NBC, K), jnp.int32),
            pltpu.VMEM((NBC, K), jnp.int32),
            pltpu.VMEM((K, H), jnp.float32),
            pltpu.VMEM_SHARED((NP, H), jnp.float32),
            pltpu.SemaphoreType.DMA,
        ],
    )


@jax.jit
def kernel(h, c, edge_index, U_iou_w, U_f_w, U_f_b, b_iou):
    src = edge_index[0].astype(jnp.int32)
    dst = edge_index[1].astype(jnp.int32)

    p = _pre(h, c, U_f_w.T, U_f_b.reshape(1, H))

    pad = EP - E
    src3 = jnp.concatenate([src, jnp.zeros((pad,), jnp.int32)])
    dst3 = jnp.concatenate(
        [dst, N + (jnp.arange(pad, dtype=jnp.int32) % (NP - N))])
    src3 = src3.reshape(NT, NQ, NBC, K)
    dst3 = dst3.reshape(NT, NQ, NBC, K)

    agg = _edge_kernel()(h, p, src3, dst3)
    h_new, c_new = _post(agg, U_iou_w.T, b_iou)
    return h_new, c_new


# speedup vs baseline: 1.1395x; 1.1238x over previous
import functools

import jax
import jax.numpy as jnp
from jax import lax
from jax.experimental import pallas as pl
from jax.experimental.pallas import tpu as pltpu
from jax.experimental.pallas import tpu_sc as plsc

N = 10000
E = 320000
H = 128

NC = 2
NT = 16
K = 80
NB = 256
EP = NT * NB * K
NQ = 8
NBC = NB // NQ
NP = 10240
RPT = NP // NT

ROW_BLK = 2000


def _pre_body(h_ref, c_ref, wt_ref, b_ref, p_ref):
    g = jax.nn.sigmoid(
        jnp.dot(h_ref[...], wt_ref[...], preferred_element_type=jnp.float32)
        + b_ref[...])
    p_ref[...] = g * c_ref[...]


_pre = pl.pallas_call(
    _pre_body,
    grid=(N // ROW_BLK,),
    in_specs=[
        pl.BlockSpec((ROW_BLK, H), lambda i: (i, 0)),
        pl.BlockSpec((ROW_BLK, H), lambda i: (i, 0)),
        pl.BlockSpec((H, H), lambda i: (0, 0)),
        pl.BlockSpec((1, H), lambda i: (0, 0)),
    ],
    out_specs=pl.BlockSpec((ROW_BLK, H), lambda i: (i, 0)),
    out_shape=jax.ShapeDtypeStruct((N, H), jnp.float32),
)


def _post_body(agg_ref, wt_ref, b_ref, h_ref, c_ref):
    ht = agg_ref[0]
    ca = agg_ref[1]
    iou = (jnp.dot(ht, wt_ref[...], preferred_element_type=jnp.float32)
           + b_ref[...])
    i = jax.nn.sigmoid(iou[:, :H])
    o = jax.nn.sigmoid(iou[:, H:2 * H])
    u = jnp.tanh(iou[:, 2 * H:])
    c_new = i * u + ca
    h_ref[...] = o * jnp.tanh(c_new)
    c_ref[...] = c_new


_post = pl.pallas_call(
    _post_body,
    grid=(N // ROW_BLK,),
    in_specs=[
        pl.BlockSpec((NC, ROW_BLK, H), lambda i: (0, i, 0)),
        pl.BlockSpec((H, 3 * H), lambda i: (0, 0)),
        pl.BlockSpec((1, 3 * H), lambda i: (0, 0)),
    ],
    out_specs=[
        pl.BlockSpec((ROW_BLK, H), lambda i: (i, 0)),
        pl.BlockSpec((ROW_BLK, H), lambda i: (i, 0)),
    ],
    out_shape=[
        jax.ShapeDtypeStruct((N, H), jnp.float32),
        jax.ShapeDtypeStruct((N, H), jnp.float32),
    ],
)


def _edge_body(tab, src3, dst3, out, src_v, dst_v, rows_v, acc, sem):
    c = lax.axis_index("c")
    s = lax.axis_index("s")

    zero16 = jnp.zeros((16,), jnp.float32)

    def _zrow(i, carry):
        for j in range(H // 16):
            rows_v[i, 16 * j:16 * (j + 1)] = zero16
        return carry

    lax.fori_loop(0, K, _zrow, 0)
    base = s * RPT
    for t in range(RPT // K):
        pltpu.sync_copy(rows_v, acc.at[pl.ds(base + K * t, K)])
    plsc.subcore_barrier()

    def _chunk(q, carry):
        pltpu.sync_copy(src3.at[c, s, q], src_v)
        pltpu.sync_copy(dst3.at[s, q], dst_v)

        def _blk(j, carry2):
            pltpu.async_copy(tab.at[src_v.at[j]], rows_v, sem).wait()
            pltpu.sync_copy(rows_v, acc.at[dst_v.at[j]], add=True)
            return carry2

        lax.fori_loop(0, NBC, _blk, 0)
        return carry

    lax.fori_loop(0, NQ, _chunk, 0)
    plsc.subcore_barrier()

    for t in range(RPT // K):
        pltpu.sync_copy(acc.at[pl.ds(base + K * t, K)], rows_v)
        pltpu.sync_copy(rows_v, out.at[c, pl.ds(base + K * t, K)])


@functools.lru_cache(maxsize=1)
def _edge_kernel():
    return pl.kernel(
        _edge_body,
        out_type=pltpu.HBM((NC, NP, H), jnp.float32),
        mesh=plsc.VectorSubcoreMesh(core_axis_name="c", subcore_axis_name="s"),
        scratch_types=[
            pltpu.VMEM((NBC, K), jnp.int32),
            pltpu.VMEM((NBC, K), jnp.int32),
            pltpu.VMEM((K, H), jnp.float32),
            pltpu.VMEM_SHARED((NP, H), jnp.float32),
            pltpu.SemaphoreType.DMA,
        ],
    )


@jax.jit
def kernel(h, c, edge_index, U_iou_w, U_f_w, U_f_b, b_iou):
    src = edge_index[0].astype(jnp.int32)
    dst = edge_index[1].astype(jnp.int32)

    p = _pre(h, c, U_f_w.T, U_f_b.reshape(1, H))

    pad = EP - E
    src_p = jnp.concatenate([src, jnp.zeros((pad,), jnp.int32)])
    dst_p = jnp.concatenate(
        [dst, N + (jnp.arange(pad, dtype=jnp.int32) % (NP - N))])
    tab = jnp.concatenate([h, p], axis=0)
    src3 = jnp.stack([src_p, src_p + N]).reshape(NC, NT, NQ, NBC, K)
    dst3 = dst_p.reshape(NT, NQ, NBC, K)

    agg = _edge_kernel()(tab, src3, dst3)
    h_new, c_new = _post(agg, U_iou_w.T, b_iou)
    return h_new, c_new


# speedup vs baseline: 1.7747x; 1.5574x over previous
import functools

import jax
import jax.numpy as jnp
from jax import lax
from jax.experimental import pallas as pl
from jax.experimental.pallas import tpu as pltpu
from jax.experimental.pallas import tpu_sc as plsc

N = 10000
E = 320000
H = 128

NC = 2
NT = 16
K = 80
NB = 250
NQ = 10
NBC = NB // NQ
NP = 10240
RPT = NP // NT

ROW_BLK = 2000


def _pre_body(h_ref, c_ref, wt_ref, b_ref, p_ref):
    g = jax.nn.sigmoid(
        jnp.dot(h_ref[...], wt_ref[...], preferred_element_type=jnp.float32)
        + b_ref[...])
    p_ref[...] = g * c_ref[...]


_pre = pl.pallas_call(
    _pre_body,
    grid=(N // ROW_BLK,),
    in_specs=[
        pl.BlockSpec((ROW_BLK, H), lambda i: (i, 0)),
        pl.BlockSpec((ROW_BLK, H), lambda i: (i, 0)),
        pl.BlockSpec((H, H), lambda i: (0, 0)),
        pl.BlockSpec((1, H), lambda i: (0, 0)),
    ],
    out_specs=pl.BlockSpec((ROW_BLK, H), lambda i: (i, 0)),
    out_shape=jax.ShapeDtypeStruct((N, H), jnp.float32),
)


def _post_body(agg_ref, wt_ref, b_ref, h_ref, c_ref):
    ht = agg_ref[0]
    ca = agg_ref[1]
    iou = (jnp.dot(ht, wt_ref[...], preferred_element_type=jnp.float32)
           + b_ref[...])
    i = jax.nn.sigmoid(iou[:, :H])
    o = jax.nn.sigmoid(iou[:, H:2 * H])
    u = jnp.tanh(iou[:, 2 * H:])
    c_new = i * u + ca
    h_ref[...] = o * jnp.tanh(c_new)
    c_ref[...] = c_new


_post = pl.pallas_call(
    _post_body,
    grid=(N // ROW_BLK,),
    in_specs=[
        pl.BlockSpec((NC, ROW_BLK, H), lambda i: (0, i, 0)),
        pl.BlockSpec((H, 3 * H), lambda i: (0, 0)),
        pl.BlockSpec((1, 3 * H), lambda i: (0, 0)),
    ],
    out_specs=[
        pl.BlockSpec((ROW_BLK, H), lambda i: (i, 0)),
        pl.BlockSpec((ROW_BLK, H), lambda i: (i, 0)),
    ],
    out_shape=[
        jax.ShapeDtypeStruct((N, H), jnp.float32),
        jax.ShapeDtypeStruct((N, H), jnp.float32),
    ],
)


def _edge_body(tab, src3, dst3, out, src_v, dst_v, rows_v, acc, sem):
    c = lax.axis_index("c")
    s = lax.axis_index("s")

    zero16 = jnp.zeros((16,), jnp.float32)

    def _zrow(i, carry):
        for j in range(H // 16):
            rows_v[i, 16 * j:16 * (j + 1)] = zero16
        return carry

    lax.fori_loop(0, K, _zrow, 0)
    base = s * RPT
    for t in range(RPT // K):
        pltpu.sync_copy(rows_v, acc.at[pl.ds(base + K * t, K)])
    plsc.subcore_barrier()

    def _chunk(q, carry):
        pltpu.sync_copy(src3.at[c, s, q], src_v)
        pltpu.sync_copy(dst3.at[s, q], dst_v)

        def _blk(j, carry2):
            pltpu.async_copy(tab.at[src_v.at[j]], rows_v, sem).wait()
            pltpu.sync_copy(rows_v, acc.at[dst_v.at[j]], add=True)
            return carry2

        lax.fori_loop(0, NBC, _blk, 0)
        return carry

    lax.fori_loop(0, NQ, _chunk, 0)
    plsc.subcore_barrier()

    for t in range(RPT // K):
        pltpu.sync_copy(acc.at[pl.ds(base + K * t, K)], rows_v)
        pltpu.sync_copy(rows_v, out.at[c, pl.ds(base + K * t, K)])


@functools.lru_cache(maxsize=1)
def _edge_kernel():
    return pl.kernel(
        _edge_body,
        out_type=pltpu.HBM((NC, NP, H), jnp.float32),
        mesh=plsc.VectorSubcoreMesh(core_axis_name="c", subcore_axis_name="s"),
        scratch_types=[
            pltpu.VMEM((NBC, K), jnp.int32),
            pltpu.VMEM((NBC, K), jnp.int32),
            pltpu.VMEM((K, H), jnp.float32),
            pltpu.VMEM_SHARED((NP, H), jnp.float32),
            pltpu.SemaphoreType.DMA,
        ],
    )


@jax.jit
def kernel(h, c, edge_index, U_iou_w, U_f_w, U_f_b, b_iou):
    src = edge_index[0].astype(jnp.int32)
    dst = edge_index[1].astype(jnp.int32)

    p = _pre(h, c, U_f_w.T, U_f_b.reshape(1, H))

    tab = jnp.concatenate([h, p], axis=0)
    src3 = jnp.stack([src, src + N]).reshape(NC, NT, NQ, NBC, K)
    dst3 = dst.reshape(NT, NQ, NBC, K)

    agg = _edge_kernel()(tab, src3, dst3)
    h_new, c_new = _post(agg, U_iou_w.T, b_iou)
    return h_new, c_new


# speedup vs baseline: 1.9605x; 1.1047x over previous
import functools

import jax
import jax.numpy as jnp
from jax import lax
from jax.experimental import pallas as pl
from jax.experimental.pallas import tpu as pltpu
from jax.experimental.pallas import tpu_sc as plsc

N = 10000
E = 320000
H = 128

NC = 2
NT = 16
K = 100
NB = 200
NQ = 2
NBC = NB // NQ
WBC = 80
NP = 10240
RPT = NP // NT

ROW_BLK = 2000


def _pre_body(h_ref, c_ref, wt_ref, b_ref, p_ref):
    g = jax.nn.sigmoid(
        jnp.dot(h_ref[...], wt_ref[...], preferred_element_type=jnp.float32)
        + b_ref[...])
    p_ref[...] = g * c_ref[...]


_pre = pl.pallas_call(
    _pre_body,
    grid=(N // ROW_BLK,),
    in_specs=[
        pl.BlockSpec((ROW_BLK, H), lambda i: (i, 0)),
        pl.BlockSpec((ROW_BLK, H), lambda i: (i, 0)),
        pl.BlockSpec((H, H), lambda i: (0, 0)),
        pl.BlockSpec((1, H), lambda i: (0, 0)),
    ],
    out_specs=pl.BlockSpec((ROW_BLK, H), lambda i: (i, 0)),
    out_shape=jax.ShapeDtypeStruct((N, H), jnp.float32),
)


def _post_body(agg_ref, wt_ref, b_ref, h_ref, c_ref):
    ht = agg_ref[0]
    ca = agg_ref[1]
    iou = (jnp.dot(ht, wt_ref[...], preferred_element_type=jnp.float32)
           + b_ref[...])
    i = jax.nn.sigmoid(iou[:, :H])
    o = jax.nn.sigmoid(iou[:, H:2 * H])
    u = jnp.tanh(iou[:, 2 * H:])
    c_new = i * u + ca
    h_ref[...] = o * jnp.tanh(c_new)
    c_ref[...] = c_new


_post = pl.pallas_call(
    _post_body,
    grid=(N // ROW_BLK,),
    in_specs=[
        pl.BlockSpec((NC, ROW_BLK, H), lambda i: (0, i, 0)),
        pl.BlockSpec((H, 3 * H), lambda i: (0, 0)),
        pl.BlockSpec((1, 3 * H), lambda i: (0, 0)),
    ],
    out_specs=[
        pl.BlockSpec((ROW_BLK, H), lambda i: (i, 0)),
        pl.BlockSpec((ROW_BLK, H), lambda i: (i, 0)),
    ],
    out_shape=[
        jax.ShapeDtypeStruct((N, H), jnp.float32),
        jax.ShapeDtypeStruct((N, H), jnp.float32),
    ],
)


def _edge_body(tab, src3, dst3, out, src_v, dst_v, rows_v, acc, sem):
    c = lax.axis_index("c")
    s = lax.axis_index("s")

    zero16 = jnp.zeros((16,), jnp.float32)

    def _zrow(i, carry):
        for j in range(H // 16):
            rows_v[i, 16 * j:16 * (j + 1)] = zero16
        return carry

    lax.fori_loop(0, WBC, _zrow, 0)
    base = s * RPT
    for t in range(RPT // WBC):
        pltpu.sync_copy(rows_v.at[pl.ds(0, WBC)],
                        acc.at[pl.ds(base + WBC * t, WBC)])
    plsc.subcore_barrier()

    def _chunk(q, carry):
        pltpu.sync_copy(src3.at[c, s, q], src_v)
        pltpu.sync_copy(dst3.at[s, q], dst_v)

        def _blk(j, carry2):
            pltpu.async_copy(tab.at[src_v.at[j]], rows_v, sem).wait()
            pltpu.sync_copy(rows_v, acc.at[dst_v.at[j]], add=True)
            return carry2

        lax.fori_loop(0, NBC, _blk, 0)
        return carry

    lax.fori_loop(0, NQ, _chunk, 0)
    plsc.subcore_barrier()

    for t in range(RPT // WBC):
        pltpu.sync_copy(acc.at[pl.ds(base + WBC * t, WBC)],
                        rows_v.at[pl.ds(0, WBC)])
        pltpu.sync_copy(rows_v.at[pl.ds(0, WBC)],
                        out.at[c, pl.ds(base + WBC * t, WBC)])


@functools.lru_cache(maxsize=1)
def _edge_kernel():
    return pl.kernel(
        _edge_body,
        out_type=pltpu.HBM((NC, NP, H), jnp.float32),
        mesh=plsc.VectorSubcoreMesh(core_axis_name="c", subcore_axis_name="s"),
        scratch_types=[
            pltpu.VMEM((NBC, K), jnp.int32),
            pltpu.VMEM((NBC, K), jnp.int32),
            pltpu.VMEM((K, H), jnp.float32),
            pltpu.VMEM_SHARED((NP, H), jnp.float32),
            pltpu.SemaphoreType.DMA,
        ],
    )


@jax.jit
def kernel(h, c, edge_index, U_iou_w, U_f_w, U_f_b, b_iou):
    src = edge_index[0].astype(jnp.int32)
    dst = edge_index[1].astype(jnp.int32)

    p = _pre(h, c, U_f_w.T, U_f_b.reshape(1, H))

    tab = jnp.concatenate([h, p], axis=0)
    src3 = jnp.stack([src, src + N]).reshape(NC, NT, NQ, NBC, K)
    dst3 = dst.reshape(NT, NQ, NBC, K)

    agg = _edge_kernel()(tab, src3, dst3)
    h_new, c_new = _post(agg, U_iou_w.T, b_iou)
    return h_new, c_new


# speedup vs baseline: 2.0796x; 1.0607x over previous
import functools

import jax
import jax.numpy as jnp
from jax import lax
from jax.experimental import pallas as pl
from jax.experimental.pallas import tpu as pltpu
from jax.experimental.pallas import tpu_sc as plsc

N = 10000
E = 320000
H = 128

NC = 2
NT = 16
K = 125
NB = 160
NQ = 2
NBC = NB // NQ
WBC = 80
NP = 10240
RPT = NP // NT

ROW_BLK = 2000


def _pre_body(h_ref, c_ref, wt_ref, b_ref, p_ref):
    g = jax.nn.sigmoid(
        jnp.dot(h_ref[...], wt_ref[...], preferred_element_type=jnp.float32)
        + b_ref[...])
    p_ref[...] = g * c_ref[...]


_pre = pl.pallas_call(
    _pre_body,
    grid=(N // ROW_BLK,),
    in_specs=[
        pl.BlockSpec((ROW_BLK, H), lambda i: (i, 0)),
        pl.BlockSpec((ROW_BLK, H), lambda i: (i, 0)),
        pl.BlockSpec((H, H), lambda i: (0, 0)),
        pl.BlockSpec((1, H), lambda i: (0, 0)),
    ],
    out_specs=pl.BlockSpec((ROW_BLK, H), lambda i: (i, 0)),
    out_shape=jax.ShapeDtypeStruct((N, H), jnp.float32),
)


def _post_body(agg_ref, wt_ref, b_ref, h_ref, c_ref):
    ht = agg_ref[0]
    ca = agg_ref[1]
    iou = (jnp.dot(ht, wt_ref[...], preferred_element_type=jnp.float32)
           + b_ref[...])
    i = jax.nn.sigmoid(iou[:, :H])
    o = jax.nn.sigmoid(iou[:, H:2 * H])
    u = jnp.tanh(iou[:, 2 * H:])
    c_new = i * u + ca
    h_ref[...] = o * jnp.tanh(c_new)
    c_ref[...] = c_new


_post = pl.pallas_call(
    _post_body,
    grid=(N // ROW_BLK,),
    in_specs=[
        pl.BlockSpec((NC, ROW_BLK, H), lambda i: (0, i, 0)),
        pl.BlockSpec((H, 3 * H), lambda i: (0, 0)),
        pl.BlockSpec((1, 3 * H), lambda i: (0, 0)),
    ],
    out_specs=[
        pl.BlockSpec((ROW_BLK, H), lambda i: (i, 0)),
        pl.BlockSpec((ROW_BLK, H), lambda i: (i, 0)),
    ],
    out_shape=[
        jax.ShapeDtypeStruct((N, H), jnp.float32),
        jax.ShapeDtypeStruct((N, H), jnp.float32),
    ],
)


def _edge_body(tab, src3, dst3, out, src_v, dst_v, rows_v, acc, sem):
    c = lax.axis_index("c")
    s = lax.axis_index("s")

    zero16 = jnp.zeros((16,), jnp.float32)

    def _zrow(i, carry):
        for j in range(H // 16):
            rows_v[i, 16 * j:16 * (j + 1)] = zero16
        return carry

    lax.fori_loop(0, WBC, _zrow, 0)
    base = s * RPT
    for t in range(RPT // WBC):
        pltpu.sync_copy(rows_v.at[pl.ds(0, WBC)],
                        acc.at[pl.ds(base + WBC * t, WBC)])
    plsc.subcore_barrier()

    def _chunk(q, carry):
        pltpu.sync_copy(src3.at[c, s, q], src_v)
        pltpu.sync_copy(dst3.at[s, q], dst_v)

        def _blk(j, carry2):
            pltpu.async_copy(tab.at[src_v.at[j]], rows_v, sem).wait()
            pltpu.sync_copy(rows_v, acc.at[dst_v.at[j]], add=True)
            return carry2

        lax.fori_loop(0, NBC, _blk, 0)
        return carry

    lax.fori_loop(0, NQ, _chunk, 0)
    plsc.subcore_barrier()

    for t in range(RPT // WBC):
        pltpu.sync_copy(acc.at[pl.ds(base + WBC * t, WBC)],
                        rows_v.at[pl.ds(0, WBC)])
        pltpu.sync_copy(rows_v.at[pl.ds(0, WBC)],
                        out.at[c, pl.ds(base + WBC * t, WBC)])


@functools.lru_cache(maxsize=1)
def _edge_kernel():
    return pl.kernel(
        _edge_body,
        out_type=pltpu.HBM((NC, NP, H), jnp.float32),
        mesh=plsc.VectorSubcoreMesh(core_axis_name="c", subcore_axis_name="s"),
        scratch_types=[
            pltpu.VMEM((NBC, K), jnp.int32),
            pltpu.VMEM((NBC, K), jnp.int32),
            pltpu.VMEM((K, H), jnp.float32),
            pltpu.VMEM_SHARED((NP, H), jnp.float32),
            pltpu.SemaphoreType.DMA,
        ],
    )


@jax.jit
def kernel(h, c, edge_index, U_iou_w, U_f_w, U_f_b, b_iou):
    src = edge_index[0].astype(jnp.int32)
    dst = edge_index[1].astype(jnp.int32)

    p = _pre(h, c, U_f_w.T, U_f_b.reshape(1, H))

    tab = jnp.concatenate([h, p], axis=0)
    src3 = jnp.stack([src, src + N]).reshape(NC, NT, NQ, NBC, K)
    dst3 = dst.reshape(NT, NQ, NBC, K)

    agg = _edge_kernel()(tab, src3, dst3)
    h_new, c_new = _post(agg, U_iou_w.T, b_iou)
    return h_new, c_new


# speedup vs baseline: 2.6622x; 1.2802x over previous
import functools

import jax
import jax.numpy as jnp
from jax import lax
from jax.experimental import pallas as pl
from jax.experimental.pallas import tpu as pltpu
from jax.experimental.pallas import tpu_sc as plsc

N = 10000
E = 320000
H = 128

NC = 2
NT = 16
K = 125
NB = 160
NQ = 4
NBC = NB // NQ
WBC = 80
NP = 10240
RPT = NP // NT

ROW_BLK = 2000


def _pre_body(h_ref, c_ref, wt_ref, b_ref, p_ref):
    g = jax.nn.sigmoid(
        jnp.dot(h_ref[...], wt_ref[...], preferred_element_type=jnp.float32)
        + b_ref[...])
    p_ref[...] = g * c_ref[...]


_pre = pl.pallas_call(
    _pre_body,
    grid=(N // ROW_BLK,),
    in_specs=[
        pl.BlockSpec((ROW_BLK, H), lambda i: (i, 0)),
        pl.BlockSpec((ROW_BLK, H), lambda i: (i, 0)),
        pl.BlockSpec((H, H), lambda i: (0, 0)),
        pl.BlockSpec((1, H), lambda i: (0, 0)),
    ],
    out_specs=pl.BlockSpec((ROW_BLK, H), lambda i: (i, 0)),
    out_shape=jax.ShapeDtypeStruct((N, H), jnp.float32),
)


def _post_body(agg_ref, wt_ref, b_ref, h_ref, c_ref):
    ht = agg_ref[0]
    ca = agg_ref[1]
    iou = (jnp.dot(ht, wt_ref[...], preferred_element_type=jnp.float32)
           + b_ref[...])
    i = jax.nn.sigmoid(iou[:, :H])
    o = jax.nn.sigmoid(iou[:, H:2 * H])
    u = jnp.tanh(iou[:, 2 * H:])
    c_new = i * u + ca
    h_ref[...] = o * jnp.tanh(c_new)
    c_ref[...] = c_new


_post = pl.pallas_call(
    _post_body,
    grid=(N // ROW_BLK,),
    in_specs=[
        pl.BlockSpec((NC, ROW_BLK, H), lambda i: (0, i, 0)),
        pl.BlockSpec((H, 3 * H), lambda i: (0, 0)),
        pl.BlockSpec((1, 3 * H), lambda i: (0, 0)),
    ],
    out_specs=[
        pl.BlockSpec((ROW_BLK, H), lambda i: (i, 0)),
        pl.BlockSpec((ROW_BLK, H), lambda i: (i, 0)),
    ],
    out_shape=[
        jax.ShapeDtypeStruct((N, H), jnp.float32),
        jax.ShapeDtypeStruct((N, H), jnp.float32),
    ],
)


def _edge_body(tab, src3, dst3, out, src_v, dst_v, r0, r1, acc, g0, g1):
    c = lax.axis_index("c")
    s = lax.axis_index("s")
    rows = (r0, r1)
    gsem = (g0, g1)
    rows_v = r0

    zero16 = jnp.zeros((16,), jnp.float32)

    def _zrow(i, carry):
        for j in range(H // 16):
            rows_v[i, 16 * j:16 * (j + 1)] = zero16
        return carry

    lax.fori_loop(0, WBC, _zrow, 0)
    base = s * RPT
    for t in range(RPT // WBC):
        pltpu.sync_copy(rows_v.at[pl.ds(0, WBC)],
                        acc.at[pl.ds(base + WBC * t, WBC)])
    plsc.subcore_barrier()

    def _gath(j, b):
        return pltpu.make_async_copy(tab.at[src_v.at[j]], rows[b], gsem[b])

    def _scat(j, b):
        pltpu.sync_copy(rows[b], acc.at[dst_v.at[j]], add=True)

    def _chunk(q, carry):
        pltpu.sync_copy(src3.at[c, s, q], src_v)
        pltpu.sync_copy(dst3.at[s, q], dst_v)
        _gath(0, 0).start()

        def _pair(i, carry2):
            j = 2 * i
            _gath(j, 0).wait()
            _gath(j + 1, 1).start()
            _scat(j, 0)
            _gath(j + 1, 1).wait()
            _gath(j + 2, 0).start()
            _scat(j + 1, 1)
            return carry2

        lax.fori_loop(0, NBC // 2 - 1, _pair, 0)
        _gath(NBC - 2, 0).wait()
        _gath(NBC - 1, 1).start()
        _scat(NBC - 2, 0)
        _gath(NBC - 1, 1).wait()
        _scat(NBC - 1, 1)
        return carry

    lax.fori_loop(0, NQ, _chunk, 0)
    plsc.subcore_barrier()

    for t in range(RPT // WBC):
        pltpu.sync_copy(acc.at[pl.ds(base + WBC * t, WBC)],
                        rows_v.at[pl.ds(0, WBC)])
        pltpu.sync_copy(rows_v.at[pl.ds(0, WBC)],
                        out.at[c, pl.ds(base + WBC * t, WBC)])


@functools.lru_cache(maxsize=1)
def _edge_kernel():
    return pl.kernel(
        _edge_body,
        out_type=pltpu.HBM((NC, NP, H), jnp.float32),
        mesh=plsc.VectorSubcoreMesh(core_axis_name="c", subcore_axis_name="s"),
        scratch_types=[
            pltpu.VMEM((NBC, K), jnp.int32),
            pltpu.VMEM((NBC, K), jnp.int32),
            pltpu.VMEM((K, H), jnp.float32),
            pltpu.VMEM((K, H), jnp.float32),
            pltpu.VMEM_SHARED((NP, H), jnp.float32),
            pltpu.SemaphoreType.DMA,
            pltpu.SemaphoreType.DMA,
        ],
    )


@jax.jit
def kernel(h, c, edge_index, U_iou_w, U_f_w, U_f_b, b_iou):
    src = edge_index[0].astype(jnp.int32)
    dst = edge_index[1].astype(jnp.int32)

    p = _pre(h, c, U_f_w.T, U_f_b.reshape(1, H))

    tab = jnp.concatenate([h, p], axis=0)
    src3 = jnp.stack([src, src + N]).reshape(NC, NT, NQ, NBC, K)
    dst3 = dst.reshape(NT, NQ, NBC, K)

    agg = _edge_kernel()(tab, src3, dst3)
    h_new, c_new = _post(agg, U_iou_w.T, b_iou)
    return h_new, c_new


# speedup vs baseline: 2.8747x; 1.0799x over previous
import functools

import jax
import jax.numpy as jnp
from jax import lax
from jax.experimental import pallas as pl
from jax.experimental.pallas import tpu as pltpu
from jax.experimental.pallas import tpu_sc as plsc

N = 10000
E = 320000
H = 128

NC = 2
NT = 16
K = 50
NB = 400
NQ = 10
NBC = NB // NQ
NBUF = 4
G = 3
WBC = 40
NP = 10240
RPT = NP // NT

ROW_BLK = 2000


def _pre_body(h_ref, c_ref, wt_ref, b_ref, p_ref):
    g = jax.nn.sigmoid(
        jnp.dot(h_ref[...], wt_ref[...], preferred_element_type=jnp.float32)
        + b_ref[...])
    p_ref[...] = g * c_ref[...]


_pre = pl.pallas_call(
    _pre_body,
    grid=(N // ROW_BLK,),
    in_specs=[
        pl.BlockSpec((ROW_BLK, H), lambda i: (i, 0)),
        pl.BlockSpec((ROW_BLK, H), lambda i: (i, 0)),
        pl.BlockSpec((H, H), lambda i: (0, 0)),
        pl.BlockSpec((1, H), lambda i: (0, 0)),
    ],
    out_specs=pl.BlockSpec((ROW_BLK, H), lambda i: (i, 0)),
    out_shape=jax.ShapeDtypeStruct((N, H), jnp.float32),
)


def _post_body(agg_ref, wt_ref, b_ref, h_ref, c_ref):
    ht = agg_ref[0]
    ca = agg_ref[1]
    iou = (jnp.dot(ht, wt_ref[...], preferred_element_type=jnp.float32)
           + b_ref[...])
    i = jax.nn.sigmoid(iou[:, :H])
    o = jax.nn.sigmoid(iou[:, H:2 * H])
    u = jnp.tanh(iou[:, 2 * H:])
    c_new = i * u + ca
    h_ref[...] = o * jnp.tanh(c_new)
    c_ref[...] = c_new


_post = pl.pallas_call(
    _post_body,
    grid=(N // ROW_BLK,),
    in_specs=[
        pl.BlockSpec((NC, ROW_BLK, H), lambda i: (0, i, 0)),
        pl.BlockSpec((H, 3 * H), lambda i: (0, 0)),
        pl.BlockSpec((1, 3 * H), lambda i: (0, 0)),
    ],
    out_specs=[
        pl.BlockSpec((ROW_BLK, H), lambda i: (i, 0)),
        pl.BlockSpec((ROW_BLK, H), lambda i: (i, 0)),
    ],
    out_shape=[
        jax.ShapeDtypeStruct((N, H), jnp.float32),
        jax.ShapeDtypeStruct((N, H), jnp.float32),
    ],
)


def _edge_body(tab, src3, dst3, out, src_v, dst_v, r0, r1, r2, r3, acc,
               g0, g1, g2, g3, s0, s1, s2, s3):
    c = lax.axis_index("c")
    s = lax.axis_index("s")
    rows = (r0, r1, r2, r3)
    gsem = (g0, g1, g2, g3)
    ssem = (s0, s1, s2, s3)
    rows_v = r0

    zero16 = jnp.zeros((16,), jnp.float32)

    def _zrow(i, carry):
        for j in range(H // 16):
            rows_v[i, 16 * j:16 * (j + 1)] = zero16
        return carry

    lax.fori_loop(0, WBC, _zrow, 0)
    base = s * RPT
    for t in range(RPT // WBC):
        pltpu.sync_copy(rows_v.at[pl.ds(0, WBC)],
                        acc.at[pl.ds(base + WBC * t, WBC)])
    plsc.subcore_barrier()

    def _gath(j, b):
        return pltpu.make_async_copy(tab.at[src_v.at[j]], rows[b], gsem[b])

    def _chunk(q, carry):
        pltpu.sync_copy(src3.at[c, s, q], src_v)
        pltpu.sync_copy(dst3.at[s, q], dst_v)
        for b in range(G):
            _gath(b, b).start()

        def _quad(i, carry2):
            for b in range(NBUF):
                j = NBUF * i + b
                _gath(j, b).wait()
                pltpu.async_copy(rows[b], acc.at[dst_v.at[j]], ssem[b],
                                 add=True)
                jn = j + G
                bn = (b + G) % NBUF

                @pl.when(jn < NBC)
                def _issue():
                    @pl.when(jn >= NBUF)
                    def _drain():
                        pltpu.make_async_copy(rows[bn],
                                              acc.at[dst_v.at[jn - NBUF]],
                                              ssem[bn]).wait()
                    _gath(jn, bn).start()
            return carry2

        lax.fori_loop(0, NBC // NBUF, _quad, 0)
        for b in range(NBUF):
            pltpu.make_async_copy(rows[b], acc.at[dst_v.at[NBC - NBUF + b]],
                                  ssem[b]).wait()
        return carry

    lax.fori_loop(0, NQ, _chunk, 0)
    plsc.subcore_barrier()

    for t in range(RPT // WBC):
        pltpu.sync_copy(acc.at[pl.ds(base + WBC * t, WBC)],
                        rows_v.at[pl.ds(0, WBC)])
        pltpu.sync_copy(rows_v.at[pl.ds(0, WBC)],
                        out.at[c, pl.ds(base + WBC * t, WBC)])


@functools.lru_cache(maxsize=1)
def _edge_kernel():
    return pl.kernel(
        _edge_body,
        out_type=pltpu.HBM((NC, NP, H), jnp.float32),
        mesh=plsc.VectorSubcoreMesh(core_axis_name="c", subcore_axis_name="s"),
        scratch_types=[
            pltpu.VMEM((NBC, K), jnp.int32),
            pltpu.VMEM((NBC, K), jnp.int32),
            pltpu.VMEM((K, H), jnp.float32),
            pltpu.VMEM((K, H), jnp.float32),
            pltpu.VMEM((K, H), jnp.float32),
            pltpu.VMEM((K, H), jnp.float32),
            pltpu.VMEM_SHARED((NP, H), jnp.float32),
            pltpu.SemaphoreType.DMA,
            pltpu.SemaphoreType.DMA,
            pltpu.SemaphoreType.DMA,
            pltpu.SemaphoreType.DMA,
            pltpu.SemaphoreType.DMA,
            pltpu.SemaphoreType.DMA,
            pltpu.SemaphoreType.DMA,
            pltpu.SemaphoreType.DMA,
        ],
    )


@jax.jit
def kernel(h, c, edge_index, U_iou_w, U_f_w, U_f_b, b_iou):
    src = edge_index[0].astype(jnp.int32)
    dst = edge_index[1].astype(jnp.int32)

    p = _pre(h, c, U_f_w.T, U_f_b.reshape(1, H))

    tab = jnp.concatenate([h, p], axis=0)
    src3 = jnp.stack([src, src + N]).reshape(NC, NT, NQ, NBC, K)
    dst3 = dst.reshape(NT, NQ, NBC, K)

    agg = _edge_kernel()(tab, src3, dst3)
    h_new, c_new = _post(agg, U_iou_w.T, b_iou)
    return h_new, c_new
